# baseline (speedup 1.0000x reference)
import functools

import jax
import jax.numpy as jnp
from jax import lax
from jax.experimental import pallas as pl
from jax.experimental.pallas import tpu as pltpu
from jax.experimental.pallas import tpu_sc as plsc

EMBED_DIM = 64
CTX = 20
LANES = 16


def _skipgram_sc(center_flat, ctx_flat, v_weight, u_weight, batch):
    info = plsc.get_sparse_core_info()
    ns = info.num_subcores
    nw = ns
    per_w = batch // nw
    chunk = 16
    n_chunks = per_w // chunk
    nrow = chunk * CTX
    assert n_chunks % 2 == 0

    mesh = plsc.VectorSubcoreMesh(
        core_axis_name="c", subcore_axis_name="s", num_cores=1)

    @functools.partial(
        pl.kernel,
        mesh=mesh,
        compiler_params=pltpu.CompilerParams(needs_layout_passes=False),
        out_type=jax.ShapeDtypeStruct((batch * CTX,), jnp.float32),
        scratch_types=(
            [
                pltpu.VMEM((per_w,), jnp.int32),
                pltpu.VMEM((per_w * CTX,), jnp.int32),
            ]
            + [pltpu.VMEM((chunk, EMBED_DIM), jnp.float32)] * 2
            + [pltpu.VMEM((nrow, EMBED_DIM), jnp.float32)] * 2
            + [pltpu.VMEM((nrow,), jnp.float32)]
            + [pltpu.SemaphoreType.DMA] * 4
        ),
    )
    def sk(center_hbm, ctx_hbm, v_hbm, u_hbm, out_hbm, *scr):
        cidx_all, uidx_all = scr[0], scr[1]
        vrows = scr[2:4]
        urows = scr[4:6]
        outb = scr[6]
        usem = scr[7:9]
        vsem = scr[9:11]
        wid = lax.axis_index("s")

        pltpu.sync_copy(center_hbm.at[pl.ds(wid * per_w, per_w)], cidx_all)
        pltpu.sync_copy(
            ctx_hbm.at[pl.ds(wid * per_w * CTX, per_w * CTX)], uidx_all)

        def fire(g, t):
            def fire_v(jj, bc):
                ivec = cidx_all[pl.ds(g * chunk + jj * LANES, LANES)]
                for k in range(LANES):
                    pltpu.async_copy(
                        v_hbm.at[ivec[k]], vrows[t].at[jj * LANES + k],
                        vsem[t])
                return bc

            def fire_u(jj, bc):
                ivec = uidx_all[pl.ds(g * nrow + jj * LANES, LANES)]
                for k in range(LANES):
                    pltpu.async_copy(
                        u_hbm.at[ivec[k]], urows[t].at[jj * LANES + k],
                        usem[t])
                return bc

            lax.fori_loop(0, chunk // LANES, fire_v, 0)
            lax.fori_loop(0, nrow // LANES, fire_u, 0)

        def wait(t):
            pltpu.make_async_copy(
                u_hbm.at[pl.ds(0, nrow)], urows[t], usem[t]).wait()
            pltpu.make_async_copy(
                v_hbm.at[pl.ds(0, chunk)], vrows[t], vsem[t]).wait()

        def compute(g, t):
            base = wid * per_w + g * chunk
            lane = lax.iota(jnp.int32, LANES)

            def grp_body(gi, bc):
                b0 = gi * 4
                vv = [[vrows[t][b0 + bb, pl.ds(k * LANES, LANES)]
                       for k in range(4)] for bb in range(4)]
                r0 = b0 * CTX
                ov = jnp.zeros((LANES,), jnp.float32)
                for r in range(4 * CTX):
                    bb = r // CTX
                    row = r0 + r
                    p = urows[t][row, pl.ds(0, LANES)] * vv[bb][0]
                    for k in range(1, 4):
                        p += urows[t][row, pl.ds(k * LANES, LANES)] * vv[bb][k]
                    s = jnp.sum(p)
                    ov = jnp.where(lane == (r % LANES), s, ov)
                    if r % LANES == LANES - 1:
                        outb[pl.ds(r0 + (r // LANES) * LANES, LANES)] = ov
                        ov = jnp.zeros((LANES,), jnp.float32)
                return bc

            lax.fori_loop(0, chunk // 4, grp_body, 0)
            pltpu.sync_copy(outb, out_hbm.at[pl.ds(base * CTX, nrow)])

        def step(g, t):
            @pl.when(g + 1 < n_chunks)
            def _():
                fire(g + 1, 1 - t)

            wait(t)
            compute(g, t)

        fire(0, 0)

        def body(h, carry):
            step(2 * h, 0)
            step(2 * h + 1, 1)
            return carry

        lax.fori_loop(0, n_chunks // 2, body, 0)

    return sk(center_flat, ctx_flat, v_weight, u_weight)


def kernel(center, contexts_and_negatives, v_weight, u_weight):
    batch = center.shape[0]
    half = batch // 2
    center_flat = center.reshape(batch).astype(jnp.int32)
    ctx_flat = contexts_and_negatives.reshape(batch * CTX).astype(jnp.int32)
    out0 = _skipgram_sc(center_flat[:half], ctx_flat[:half * CTX],
                        v_weight, u_weight, half)
    out1 = _skipgram_sc(center_flat[half:], ctx_flat[half * CTX:],
                        v_weight, u_weight, half)
    out = jnp.concatenate([out0, out1])
    return out.reshape(batch, 1, CTX)


# speedup vs baseline: 1.0833x; 1.0833x over previous
import functools

import jax
import jax.numpy as jnp
from jax import lax
from jax.experimental import pallas as pl
from jax.experimental.pallas import tpu as pltpu
from jax.experimental.pallas import tpu_sc as plsc

EMBED_DIM = 64
CTX = 20
LANES = 16


def _skipgram_sc(center_flat, ctx_flat, v_weight, u_weight, batch):
    info = plsc.get_sparse_core_info()
    nc, ns = info.num_cores, info.num_subcores
    nw = nc * ns
    per_w = batch // nw
    chunk = 16
    n_chunks = per_w // chunk
    nrow = chunk * CTX
    assert n_chunks % 2 == 0

    mesh = plsc.VectorSubcoreMesh(core_axis_name="c", subcore_axis_name="s")

    @functools.partial(
        pl.kernel,
        mesh=mesh,
        compiler_params=pltpu.CompilerParams(needs_layout_passes=False),
        out_type=jax.ShapeDtypeStruct((batch * CTX,), jnp.float32),
        scratch_types=(
            [
                pltpu.VMEM((per_w,), jnp.int32),
                pltpu.VMEM((per_w * CTX,), jnp.int32),
            ]
            + [pltpu.VMEM((chunk, EMBED_DIM), jnp.float32)] * 2
            + [pltpu.VMEM((nrow, EMBED_DIM), jnp.float32)] * 2
            + [pltpu.VMEM((nrow,), jnp.float32)]
            + [pltpu.SemaphoreType.DMA] * 4
        ),
    )
    def sk(center_hbm, ctx_hbm, v_hbm, u_hbm, out_hbm, *scr):
        cidx_all, uidx_all = scr[0], scr[1]
        vrows = scr[2:4]
        urows = scr[4:6]
        outb = scr[6]
        usem = scr[7:9]
        vsem = scr[9:11]
        wid = lax.axis_index("s") * nc + lax.axis_index("c")

        pltpu.sync_copy(center_hbm.at[pl.ds(wid * per_w, per_w)], cidx_all)
        pltpu.sync_copy(
            ctx_hbm.at[pl.ds(wid * per_w * CTX, per_w * CTX)], uidx_all)

        def fire(g, t):
            def fire_v(jj, bc):
                ivec = cidx_all[pl.ds(g * chunk + jj * LANES, LANES)]
                for k in range(LANES):
                    pltpu.async_copy(
                        v_hbm.at[ivec[k]], vrows[t].at[jj * LANES + k],
                        vsem[t])
                return bc

            def fire_u(jj, bc):
                ivec = uidx_all[pl.ds(g * nrow + jj * LANES, LANES)]
                for k in range(LANES):
                    pltpu.async_copy(
                        u_hbm.at[ivec[k]], urows[t].at[jj * LANES + k],
                        usem[t])
                return bc

            lax.fori_loop(0, chunk // LANES, fire_v, 0)
            lax.fori_loop(0, nrow // LANES, fire_u, 0)

        def wait(t):
            pltpu.make_async_copy(
                u_hbm.at[pl.ds(0, nrow)], urows[t], usem[t]).wait()
            pltpu.make_async_copy(
                v_hbm.at[pl.ds(0, chunk)], vrows[t], vsem[t]).wait()

        def compute(g, t):
            base = wid * per_w + g * chunk
            lane = lax.iota(jnp.int32, LANES)

            def grp_body(gi, bc):
                b0 = gi * 4
                vv = [[vrows[t][b0 + bb, pl.ds(k * LANES, LANES)]
                       for k in range(4)] for bb in range(4)]
                r0 = b0 * CTX
                ov = jnp.zeros((LANES,), jnp.float32)
                for r in range(4 * CTX):
                    bb = r // CTX
                    row = r0 + r
                    p = urows[t][row, pl.ds(0, LANES)] * vv[bb][0]
                    for k in range(1, 4):
                        p += urows[t][row, pl.ds(k * LANES, LANES)] * vv[bb][k]
                    s = jnp.sum(p)
                    ov = jnp.where(lane == (r % LANES), s, ov)
                    if r % LANES == LANES - 1:
                        outb[pl.ds(r0 + (r // LANES) * LANES, LANES)] = ov
                        ov = jnp.zeros((LANES,), jnp.float32)
                return bc

            lax.fori_loop(0, chunk // 4, grp_body, 0)
            pltpu.sync_copy(outb, out_hbm.at[pl.ds(base * CTX, nrow)])

        def step(g, t):
            @pl.when(g + 1 < n_chunks)
            def _():
                fire(g + 1, 1 - t)

            wait(t)
            compute(g, t)

        fire(0, 0)

        def body(h, carry):
            step(2 * h, 0)
            step(2 * h + 1, 1)
            return carry

        lax.fori_loop(0, n_chunks // 2, body, 0)

    return sk(center_flat, ctx_flat, v_weight, u_weight)


def kernel(center, contexts_and_negatives, v_weight, u_weight):
    batch = center.shape[0]
    center_flat = center.reshape(batch).astype(jnp.int32)
    ctx_flat = contexts_and_negatives.reshape(batch * CTX).astype(jnp.int32)
    out = _skipgram_sc(center_flat, ctx_flat, v_weight, u_weight, batch)
    return out.reshape(batch, 1, CTX)
